# baseline (device time: 9642 ns/iter reference)
import jax
import jax.numpy as jnp
from jax import lax
from jax.experimental import pallas as pl
from jax.experimental.pallas import tpu as pltpu

N_DEV = 16
N_STEPS = 4


def kernel(x):
    m, n = x.shape

    def body(x_ref, out_ref, run_ref, inbox_ref, send_sems, recv_sems):
        my = lax.axis_index("i")

        barrier = pltpu.get_barrier_semaphore()
        cnt = jnp.int32(0)
        for s in range(N_STEPS):
            d = 1 << s
            lo = my - d
            hi = my + d

            @pl.when(lo >= 0)
            def _():
                pl.semaphore_signal(
                    barrier, inc=1, device_id=(lo,),
                    device_id_type=pl.DeviceIdType.MESH,
                )

            @pl.when(hi < N_DEV)
            def _():
                pl.semaphore_signal(
                    barrier, inc=1, device_id=(jnp.minimum(hi, N_DEV - 1),),
                    device_id_type=pl.DeviceIdType.MESH,
                )

            cnt = cnt + (lo >= 0).astype(jnp.int32) + (hi < N_DEV).astype(
                jnp.int32
            )
        pl.semaphore_wait(barrier, cnt)

        total = jnp.sum(x_ref[:, :], axis=0, keepdims=True)
        run_ref[0:1, :] = total

        rdmas = []
        for s in range(N_STEPS):
            d = 1 << s
            rdmas.append(
                pltpu.make_async_remote_copy(
                    src_ref=run_ref,
                    dst_ref=inbox_ref.at[s],
                    send_sem=send_sems.at[s],
                    recv_sem=recv_sems.at[s],
                    device_id=(jnp.minimum(my + d, N_DEV - 1),),
                    device_id_type=pl.DeviceIdType.MESH,
                )
            )

        @pl.when(my + 1 < N_DEV)
        def _():
            rdmas[0].start()

        acc = x_ref[:, :]
        d = 1
        while d < m:
            shifted = jnp.concatenate(
                [jnp.zeros((d, n), jnp.float32), acc[: m - d, :]], axis=0
            )
            acc = acc + shifted
            d *= 2
        out_ref[:, :] = acc

        for s in range(N_STEPS):
            d = 1 << s
            has_src = my - d >= 0
            has_dst = my + d < N_DEV
            if s > 0:

                @pl.when(has_dst)
                def _():
                    rdmas[s].start()

            @pl.when(has_src)
            def _():
                rdmas[s].wait_recv()

            @pl.when(has_dst)
            def _():
                rdmas[s].wait_send()

            @pl.when(has_src)
            def _():
                run_ref[0:1, :] = run_ref[0:1, :] + inbox_ref[s]

        out_ref[:, :] = out_ref[:, :] + (run_ref[0:1, :] - total)

    return pl.pallas_call(
        body,
        out_shape=jax.ShapeDtypeStruct((m, n), jnp.float32),
        in_specs=[pl.BlockSpec(memory_space=pltpu.VMEM)],
        out_specs=pl.BlockSpec(memory_space=pltpu.VMEM),
        scratch_shapes=[
            pltpu.VMEM((1, n), jnp.float32),
            pltpu.VMEM((N_STEPS, 1, n), jnp.float32),
            pltpu.SemaphoreType.DMA((N_STEPS,)),
            pltpu.SemaphoreType.DMA((N_STEPS,)),
        ],
        compiler_params=pltpu.CompilerParams(collective_id=0),
    )(x)
